# baseline (speedup 1.0000x reference)
import jax
import jax.numpy as jnp
from jax import lax
from jax.experimental import pallas as pl
from jax.experimental.pallas import tpu as pltpu
from jax.experimental.pallas import tpu_sc as plsc

import functools


def _make_sc_kernel(B, S, W):
    info = plsc.get_sparse_core_info()
    NC, NS, L = info.num_cores, info.num_subcores, info.num_lanes
    NW = NC * NS
    mid = S // 2
    assert mid % NW == 0
    K = mid // NW
    assert K % L == 0

    mesh = plsc.VectorSubcoreMesh(core_axis_name="c", subcore_axis_name="s")

    @functools.partial(
        pl.kernel,
        out_type=jax.ShapeDtypeStruct((B * S, W), jnp.float32),
        mesh=mesh,
        scratch_types=[
            pltpu.VMEM((K, W), jnp.float32),
            pltpu.VMEM((2 * B, K // 2), jnp.int32),
            pltpu.VMEM((1, W), jnp.float32),
            pltpu.SemaphoreType.DMA,
            pltpu.SemaphoreType.DMA,
        ],
    )
    def k(emb_hbm, out_hbm, buf, idx, x0, sem, gsem):
        wid = lax.axis_index("s") * NC + lax.axis_index("c")
        d0 = wid * K
        last = wid == NW - 1
        H = K // 2

        pltpu.async_copy(emb_hbm.at[pl.ds(d0, H)], buf.at[pl.ds(0, H)], gsem)
        pltpu.async_copy(emb_hbm.at[pl.ds(d0 + H, H)], buf.at[pl.ds(H, H)], gsem)

        @pl.when(last)
        def _():
            pltpu.async_copy(emb_hbm.at[pl.ds(mid, 1)], x0, gsem)

        for b in range(B):
            for h in range(2):
                for c in range(H // L):
                    base = b * S + mid - d0 - h * H - c * L
                    idx[2 * b + h, pl.ds(c * L, L)] = base - lax.iota(jnp.int32, L)

        copies = []
        for h in range(2):
            pltpu.make_async_copy(
                emb_hbm.at[pl.ds(d0 + h * H, H)], buf.at[pl.ds(h * H, H)], gsem
            ).wait()
            src_h = buf.at[pl.ds(h * H, H)]
            for b in range(B):
                copies.append(
                    pltpu.async_copy(
                        emb_hbm.at[pl.ds(d0 + h * H, H)],
                        out_hbm.at[pl.ds(b * S + mid + d0 + h * H, H)],
                        sem,
                    )
                )
                copies.append(pltpu.async_copy(src_h, out_hbm.at[idx.at[2 * b + h]], sem))

        @pl.when(last)
        def _():
            pltpu.make_async_copy(emb_hbm.at[pl.ds(mid, 1)], x0, gsem).wait()
            for b in range(B):
                pltpu.async_copy(x0, out_hbm.at[pl.ds(b * S, 1)], sem).wait()

        for cp in copies:
            cp.wait()

    return k


def kernel(inputs, embeddings):
    B, S, W = inputs.shape
    out = _make_sc_kernel(B, S, W)(embeddings)
    return out.reshape(B, S, W)


# speedup vs baseline: 24.1768x; 24.1768x over previous
import jax
import jax.numpy as jnp
from jax import lax
from jax.experimental import pallas as pl
from jax.experimental.pallas import tpu as pltpu
from jax.experimental.pallas import tpu_sc as plsc

import functools


def _make_sc_kernel(B, S, W):
    info = plsc.get_sparse_core_info()
    NC, NS, L = info.num_cores, info.num_subcores, info.num_lanes
    NW = NC * NS
    mid = S // 2
    assert mid % NW == 0
    K = mid // NW
    assert K % L == 0

    mesh = plsc.VectorSubcoreMesh(core_axis_name="c", subcore_axis_name="s")

    @functools.partial(
        pl.kernel,
        out_type=jax.ShapeDtypeStruct((B * S, W), jnp.float32),
        mesh=mesh,
        scratch_types=[
            pltpu.VMEM((K, W), jnp.float32),
            pltpu.VMEM((2 * B, K // 2), jnp.int32),
            pltpu.VMEM((1, W), jnp.float32),
            pltpu.SemaphoreType.DMA,
            pltpu.SemaphoreType.DMA,
        ],
    )
    def k(emb_hbm, out_hbm, buf, idx, x0, sem, gsem):
        wid = lax.axis_index("s") * NC + lax.axis_index("c")
        d0 = wid * K
        last = wid == NW - 1
        H = K // 2

        pltpu.async_copy(emb_hbm.at[pl.ds(d0, H)], buf.at[pl.ds(0, H)], gsem)
        pltpu.async_copy(emb_hbm.at[pl.ds(d0 + H, H)], buf.at[pl.ds(H, H)], gsem)

        @pl.when(last)
        def _():
            pltpu.async_copy(emb_hbm.at[pl.ds(mid, 1)], x0, gsem)

        for b in range(B):
            for h in range(2):
                for c in range(H // L):
                    base = b * S + mid - d0 - h * H - c * L
                    idx[2 * b + h, pl.ds(c * L, L)] = base - lax.iota(jnp.int32, L)

        copies = []
        for h in range(2):
            pltpu.make_async_copy(
                emb_hbm.at[pl.ds(d0 + h * H, H)], buf.at[pl.ds(h * H, H)], gsem
            ).wait()
            src_h = buf.at[pl.ds(h * H, H)]
            for b in range(B):
                copies.append(
                    pltpu.async_copy(
                        src_h, out_hbm.at[pl.ds(b * S + mid + d0 + h * H, H)], sem
                    )
                )
                copies.append(pltpu.async_copy(src_h, out_hbm.at[idx.at[2 * b + h]], sem))

        @pl.when(last)
        def _():
            pltpu.make_async_copy(emb_hbm.at[pl.ds(mid, 1)], x0, gsem).wait()
            for b in range(B):
                pltpu.async_copy(x0, out_hbm.at[pl.ds(b * S, 1)], sem).wait()

        for cp in copies:
            cp.wait()

    return k


def kernel(inputs, embeddings):
    B, S, W = inputs.shape
    out = _make_sc_kernel(B, S, W)(embeddings)
    return out.reshape(B, S, W)
